# baseline (speedup 1.0000x reference)
import functools

import jax
import jax.numpy as jnp
from jax import lax
from jax.experimental import pallas as pl
from jax.experimental.pallas import tpu as pltpu
from jax.experimental.pallas import tpu_sc as plsc

N = 10000
E = 320000
D = 128

NC = 2
NS = 16
NW = NC * NS

CH = 128
NCHUNK = E // CH
MAX_CHUNKS_PER_TILE = (NCHUNK + NW - 1) // NW
NP_ = 10240
ROWS_PER_TILE = NP_ // NS
ZR = 128

_MESH = plsc.VectorSubcoreMesh(core_axis_name="c", subcore_axis_name="s")


def _zero_fill(buf, nrows):
    z16 = jnp.zeros((16,), jnp.float32)

    def fill(r, _):
        for cb in range(D // 16):
            buf[r, pl.ds(cb * 16, 16)] = z16
        return 0
    lax.fori_loop(0, nrows, fill, 0)


def _agg_body(x_hbm, src_hbm, dst_hbm, part_hbm, agg_sh,
              s0, s1, s2, s3, d0, d1, d2, d3, rows0, rows1,
              gsem0, gsem1, isem0, isem1):
    cid = lax.axis_index("c")
    sid = lax.axis_index("s")
    wid = sid * NC + cid
    S = [s0, s1, s2, s3]
    Dx = [d0, d1, d2, d3]
    R = [rows0, rows1]
    G = [gsem0, gsem1]
    I = [isem0, isem1]

    _zero_fill(rows0, ZR)
    zbase = sid * ROWS_PER_TILE
    for k in range(ROWS_PER_TILE // ZR):
        pltpu.sync_copy(rows0, agg_sh.at[pl.ds(zbase + k * ZR, ZR)])
    plsc.subcore_barrier()

    def jc(i):
        return jnp.minimum(wid + i * NW, NCHUNK - 1)

    def idx_sync(i, q):
        b = jc(i) * CH
        pltpu.sync_copy(src_hbm.at[pl.ds(b, CH)], S[q])
        pltpu.sync_copy(dst_hbm.at[pl.ds(b, CH)], Dx[q])

    def idx_async(i, q, sem):
        b = jc(i) * CH
        pltpu.async_copy(src_hbm.at[pl.ds(b, CH)], S[q], sem)
        pltpu.async_copy(dst_hbm.at[pl.ds(b, CH)], Dx[q], sem)

    def idx_drain(q, sem):
        pltpu.make_async_copy(src_hbm.at[pl.ds(0, CH)], S[q], sem).wait()
        pltpu.make_async_copy(dst_hbm.at[pl.ds(0, CH)], Dx[q], sem).wait()

    def gather_start(q, b):
        pltpu.async_copy(x_hbm.at[S[q]], R[b], G[b])

    def gather_drain(q, b):
        pltpu.make_async_copy(x_hbm.at[S[q]], R[b], G[b]).wait()

    idx_sync(0, 0)
    gather_start(0, 0)
    idx_sync(1, 1)
    gather_start(1, 1)
    idx_async(2, 2, I[0])

    def quad(i4, _):
        i0 = i4 * 4
        for u in range(4):
            i = i0 + u
            b = u % 2
            qn = (u + 2) % 4
            idx_async(i + 3, (u + 3) % 4, I[(u + 3) % 2])
            gather_drain(u, b)

            @pl.when(wid + i * NW < NCHUNK)
            def _():
                pltpu.sync_copy(R[b], agg_sh.at[Dx[u]], add=True)

            idx_drain(qn, I[u % 2])
            gather_start(qn, b)
        return 0
    lax.fori_loop(0, MAX_CHUNKS_PER_TILE // 4 + 1, quad, 0)

    gather_drain(0, 0)
    gather_drain(1, 1)
    idx_drain(2, I[0])
    plsc.subcore_barrier()

    for k in range(ROWS_PER_TILE // ZR):
        r0 = zbase + k * ZR
        pltpu.sync_copy(agg_sh.at[pl.ds(r0, ZR)],
                        part_hbm.at[cid, pl.ds(r0, ZR)])


_sc_agg = pl.kernel(
    _agg_body,
    out_type=(jax.ShapeDtypeStruct((NC, NP_, D), jnp.float32),),
    mesh=_MESH,
    scratch_types=[
        pltpu.VMEM_SHARED((NP_, D), jnp.float32),
        pltpu.VMEM((CH,), jnp.int32),
        pltpu.VMEM((CH,), jnp.int32),
        pltpu.VMEM((CH,), jnp.int32),
        pltpu.VMEM((CH,), jnp.int32),
        pltpu.VMEM((CH,), jnp.int32),
        pltpu.VMEM((CH,), jnp.int32),
        pltpu.VMEM((CH,), jnp.int32),
        pltpu.VMEM((CH,), jnp.int32),
        pltpu.VMEM((CH, D), jnp.float32),
        pltpu.VMEM((CH, D), jnp.float32),
        pltpu.SemaphoreType.DMA,
        pltpu.SemaphoreType.DMA,
        pltpu.SemaphoreType.DMA,
        pltpu.SemaphoreType.DMA,
    ],
)


def _cnt_body(dst_hbm, cnt_hbm, cnt_sh, dst_idx, ones_v):
    cid = lax.axis_index("c")
    sid = lax.axis_index("s")
    wid = sid * NC + cid
    o16 = jnp.ones((16,), jnp.float32)

    _zero_fill(ones_v, ZR)
    zbase = sid * ROWS_PER_TILE
    for k in range(ROWS_PER_TILE // ZR):
        pltpu.sync_copy(ones_v, cnt_sh.at[pl.ds(zbase + k * ZR, ZR)])

    def fill_ones(r, _):
        for cb in range(D // 16):
            ones_v[r, pl.ds(cb * 16, 16)] = o16
        return 0
    lax.fori_loop(0, CH, fill_ones, 0)
    plsc.subcore_barrier()

    def chunk_body(i, _):
        j = wid + i * NW

        @pl.when(j < NCHUNK)
        def _():
            base_e = j * CH
            pltpu.sync_copy(dst_hbm.at[pl.ds(base_e, CH)], dst_idx)
            pltpu.sync_copy(ones_v, cnt_sh.at[dst_idx], add=True)
        return 0
    lax.fori_loop(0, MAX_CHUNKS_PER_TILE, chunk_body, 0)
    plsc.subcore_barrier()

    for k in range(ROWS_PER_TILE // ZR):
        r0 = zbase + k * ZR
        pltpu.sync_copy(cnt_sh.at[pl.ds(r0, ZR)],
                        cnt_hbm.at[cid, pl.ds(r0, ZR)])


_sc_cnt = pl.kernel(
    _cnt_body,
    out_type=(jax.ShapeDtypeStruct((NC, NP_, D), jnp.float32),),
    mesh=_MESH,
    scratch_types=[
        pltpu.VMEM_SHARED((NP_, D), jnp.float32),
        pltpu.VMEM((CH,), jnp.int32),
        pltpu.VMEM((CH, D), jnp.float32),
    ],
)


def _tc_body(part_ref, cnt_ref, x_ref, wl_ref, wr_ref, b_ref, out_ref, *,
             relu):
    p = part_ref[0] + part_ref[1]
    c = cnt_ref[0] + cnt_ref[1]
    agg = p / jnp.maximum(c, 1.0)
    y = (jnp.dot(agg, wl_ref[...], preferred_element_type=jnp.float32)
         + jnp.dot(x_ref[...], wr_ref[...], preferred_element_type=jnp.float32)
         + b_ref[...])
    if relu:
        y = jnp.maximum(y, 0.0)
    out_ref[...] = y


def _tc_layer(part, cnt, x, wl_t, wr_t, b, relu):
    BR = 1000
    grid = (N // BR,)
    return pl.pallas_call(
        functools.partial(_tc_body, relu=relu),
        grid=grid,
        in_specs=[
            pl.BlockSpec((NC, BR, D), lambda i: (0, i, 0)),
            pl.BlockSpec((NC, BR, D), lambda i: (0, i, 0)),
            pl.BlockSpec((BR, D), lambda i: (i, 0)),
            pl.BlockSpec((D, D), lambda i: (0, 0)),
            pl.BlockSpec((D, D), lambda i: (0, 0)),
            pl.BlockSpec((1, D), lambda i: (0, 0)),
        ],
        out_specs=pl.BlockSpec((BR, D), lambda i: (i, 0)),
        out_shape=jax.ShapeDtypeStruct((N, D), jnp.float32),
    )(part, cnt, x, wl_t, wr_t, b)


def kernel(x, edge_index, W1_l, b1, W1_r, W2_l, b2, W2_r):
    src = edge_index[0]
    dst = edge_index[1]

    (cnt,) = _sc_cnt(dst)
    (part1,) = _sc_agg(x, src, dst)
    h = _tc_layer(part1, cnt, x, W1_l.T, W1_r.T, b1[None, :], relu=True)
    (part2,) = _sc_agg(h, src, dst)
    out = _tc_layer(part2, cnt, h, W2_l.T, W2_r.T, b2[None, :], relu=False)
    return out


# speedup vs baseline: 1.0844x; 1.0844x over previous
import functools

import jax
import jax.numpy as jnp
from jax import lax
from jax.experimental import pallas as pl
from jax.experimental.pallas import tpu as pltpu
from jax.experimental.pallas import tpu_sc as plsc

N = 10000
E = 320000
D = 128

NC = 2
NS = 16
NW = NC * NS

CH = 128
NCHUNK = E // CH
MAX_CHUNKS_PER_TILE = (NCHUNK + NW - 1) // NW
NP_ = 10240
ROWS_PER_TILE = NP_ // NS
ZR = 128

_MESH = plsc.VectorSubcoreMesh(core_axis_name="c", subcore_axis_name="s")


def _zero_fill(buf, nrows):
    z16 = jnp.zeros((16,), jnp.float32)

    def fill(r, _):
        for cb in range(D // 16):
            buf[r, pl.ds(cb * 16, 16)] = z16
        return 0
    lax.fori_loop(0, nrows, fill, 0)


def _agg_body(x_hbm, src_hbm, dst_hbm, part_hbm, agg_sh,
              s0, s1, s2, s3, d0, d1, d2, d3, rows0, rows1,
              gsem0, gsem1, isem0, isem1):
    cid = lax.axis_index("c")
    sid = lax.axis_index("s")
    wid = sid * NC + cid
    S = [s0, s1, s2, s3]
    Dx = [d0, d1, d2, d3]
    R = [rows0, rows1]
    G = [gsem0, gsem1]
    I = [isem0, isem1]

    _zero_fill(rows0, ZR)
    zbase = sid * ROWS_PER_TILE
    for k in range(ROWS_PER_TILE // ZR):
        pltpu.sync_copy(rows0, agg_sh.at[pl.ds(zbase + k * ZR, ZR)])
    plsc.subcore_barrier()

    def jc(i):
        return jnp.minimum(wid + i * NW, NCHUNK - 1)

    def idx_sync(i, q):
        b = jc(i) * CH
        pltpu.sync_copy(src_hbm.at[pl.ds(b, CH)], S[q])
        pltpu.sync_copy(dst_hbm.at[pl.ds(b, CH)], Dx[q])

    def idx_async(i, q, sem):
        b = jc(i) * CH
        pltpu.async_copy(src_hbm.at[pl.ds(b, CH)], S[q], sem)
        pltpu.async_copy(dst_hbm.at[pl.ds(b, CH)], Dx[q], sem)

    def idx_drain(q, sem):
        pltpu.make_async_copy(src_hbm.at[pl.ds(0, CH)], S[q], sem).wait()
        pltpu.make_async_copy(dst_hbm.at[pl.ds(0, CH)], Dx[q], sem).wait()

    def gather_start(q, b):
        pltpu.async_copy(x_hbm.at[S[q]], R[b], G[b])

    def gather_drain(q, b):
        pltpu.make_async_copy(x_hbm.at[S[q]], R[b], G[b]).wait()

    idx_sync(0, 0)
    gather_start(0, 0)
    idx_sync(1, 1)
    gather_start(1, 1)
    idx_async(2, 2, I[0])

    def quad(i4, _):
        i0 = i4 * 4
        for u in range(4):
            i = i0 + u
            b = u % 2
            qn = (u + 2) % 4
            idx_async(i + 3, (u + 3) % 4, I[(u + 3) % 2])
            gather_drain(u, b)

            @pl.when(wid + i * NW < NCHUNK)
            def _():
                pltpu.sync_copy(R[b], agg_sh.at[Dx[u]], add=True)

            idx_drain(qn, I[u % 2])
            gather_start(qn, b)
        return 0
    lax.fori_loop(0, MAX_CHUNKS_PER_TILE // 4 + 1, quad, 0)

    gather_drain(0, 0)
    gather_drain(1, 1)
    idx_drain(2, I[0])
    plsc.subcore_barrier()

    for k in range(ROWS_PER_TILE // ZR):
        r0 = zbase + k * ZR
        pltpu.sync_copy(agg_sh.at[pl.ds(r0, ZR)],
                        part_hbm.at[cid, pl.ds(r0, ZR)])


_sc_agg = pl.kernel(
    _agg_body,
    out_type=(jax.ShapeDtypeStruct((NC, NP_, D), jnp.float32),),
    mesh=_MESH,
    scratch_types=[
        pltpu.VMEM_SHARED((NP_, D), jnp.float32),
        pltpu.VMEM((CH,), jnp.int32),
        pltpu.VMEM((CH,), jnp.int32),
        pltpu.VMEM((CH,), jnp.int32),
        pltpu.VMEM((CH,), jnp.int32),
        pltpu.VMEM((CH,), jnp.int32),
        pltpu.VMEM((CH,), jnp.int32),
        pltpu.VMEM((CH,), jnp.int32),
        pltpu.VMEM((CH,), jnp.int32),
        pltpu.VMEM((CH, D), jnp.float32),
        pltpu.VMEM((CH, D), jnp.float32),
        pltpu.SemaphoreType.DMA,
        pltpu.SemaphoreType.DMA,
        pltpu.SemaphoreType.DMA,
        pltpu.SemaphoreType.DMA,
    ],
)


def _cnt_body(dst_hbm, cnt_hbm, cnt_sh, d0, d1, d2, d3, ones_v,
              isem0, isem1):
    cid = lax.axis_index("c")
    sid = lax.axis_index("s")
    wid = sid * NC + cid
    o16 = jnp.ones((16,), jnp.float32)
    Dx = [d0, d1, d2, d3]
    I = [isem0, isem1]

    _zero_fill(ones_v, ZR)
    zbase = sid * ROWS_PER_TILE
    for k in range(ROWS_PER_TILE // ZR):
        pltpu.sync_copy(ones_v, cnt_sh.at[pl.ds(zbase + k * ZR, ZR)])

    def fill_ones(r, _):
        for cb in range(D // 16):
            ones_v[r, pl.ds(cb * 16, 16)] = o16
        return 0
    lax.fori_loop(0, CH, fill_ones, 0)
    plsc.subcore_barrier()

    def jc(i):
        return jnp.minimum(wid + i * NW, NCHUNK - 1)

    def idx_async(i, q, sem):
        pltpu.async_copy(dst_hbm.at[pl.ds(jc(i) * CH, CH)], Dx[q], sem)

    def idx_drain(q, sem):
        pltpu.make_async_copy(dst_hbm.at[pl.ds(0, CH)], Dx[q], sem).wait()

    idx_async(0, 0, I[0])
    idx_async(1, 1, I[1])

    def quad(i4, _):
        i0 = i4 * 4
        for u in range(4):
            i = i0 + u
            idx_drain(u, I[u % 2])
            idx_async(i + 2, (u + 2) % 4, I[u % 2])

            @pl.when(wid + i * NW < NCHUNK)
            def _():
                pltpu.sync_copy(ones_v, cnt_sh.at[Dx[u]], add=True)
        return 0
    lax.fori_loop(0, MAX_CHUNKS_PER_TILE // 4 + 1, quad, 0)

    idx_drain(0, I[0])
    idx_drain(1, I[1])
    plsc.subcore_barrier()

    for k in range(ROWS_PER_TILE // ZR):
        r0 = zbase + k * ZR
        pltpu.sync_copy(cnt_sh.at[pl.ds(r0, ZR)],
                        cnt_hbm.at[cid, pl.ds(r0, ZR)])


_sc_cnt = pl.kernel(
    _cnt_body,
    out_type=(jax.ShapeDtypeStruct((NC, NP_, D), jnp.float32),),
    mesh=_MESH,
    scratch_types=[
        pltpu.VMEM_SHARED((NP_, D), jnp.float32),
        pltpu.VMEM((CH,), jnp.int32),
        pltpu.VMEM((CH,), jnp.int32),
        pltpu.VMEM((CH,), jnp.int32),
        pltpu.VMEM((CH,), jnp.int32),
        pltpu.VMEM((CH, D), jnp.float32),
        pltpu.SemaphoreType.DMA,
        pltpu.SemaphoreType.DMA,
    ],
)


def _tc_body(part_ref, cnt_ref, x_ref, wl_ref, wr_ref, b_ref, out_ref, *,
             relu):
    p = part_ref[0] + part_ref[1]
    c = cnt_ref[0] + cnt_ref[1]
    agg = p / jnp.maximum(c, 1.0)
    y = (jnp.dot(agg, wl_ref[...], preferred_element_type=jnp.float32)
         + jnp.dot(x_ref[...], wr_ref[...], preferred_element_type=jnp.float32)
         + b_ref[...])
    if relu:
        y = jnp.maximum(y, 0.0)
    out_ref[...] = y


def _tc_layer(part, cnt, x, wl_t, wr_t, b, relu):
    BR = 1000
    grid = (N // BR,)
    return pl.pallas_call(
        functools.partial(_tc_body, relu=relu),
        grid=grid,
        in_specs=[
            pl.BlockSpec((NC, BR, D), lambda i: (0, i, 0)),
            pl.BlockSpec((NC, BR, D), lambda i: (0, i, 0)),
            pl.BlockSpec((BR, D), lambda i: (i, 0)),
            pl.BlockSpec((D, D), lambda i: (0, 0)),
            pl.BlockSpec((D, D), lambda i: (0, 0)),
            pl.BlockSpec((1, D), lambda i: (0, 0)),
        ],
        out_specs=pl.BlockSpec((BR, D), lambda i: (i, 0)),
        out_shape=jax.ShapeDtypeStruct((N, D), jnp.float32),
    )(part, cnt, x, wl_t, wr_t, b)


def kernel(x, edge_index, W1_l, b1, W1_r, W2_l, b2, W2_r):
    src = edge_index[0]
    dst = edge_index[1]

    (cnt,) = _sc_cnt(dst)
    (part1,) = _sc_agg(x, src, dst)
    h = _tc_layer(part1, cnt, x, W1_l.T, W1_r.T, b1[None, :], relu=True)
    (part2,) = _sc_agg(h, src, dst)
    out = _tc_layer(part2, cnt, h, W2_l.T, W2_r.T, b2[None, :], relu=False)
    return out
